# baseline (speedup 1.0000x reference)
import functools

import jax
import jax.numpy as jnp
from jax import lax
from jax.experimental import pallas as pl
from jax.experimental.pallas import tpu as pltpu
from jax.experimental.pallas import tpu_sc as plsc

N = 10000
MAX_LEN = 32
D = 128
B = 4096
NPAD = 10240
ROWS_BLK = 1024
NW = 32
BPW = B // NW
CHUNK_NODES = 4
NCHUNKS = BPW // CHUNK_NODES


def _project_body(x_ref, wa_ref, wb_ref, fa_ref, fb_ref):
    i = pl.program_id(0)
    row = i * ROWS_BLK + lax.broadcasted_iota(jnp.int32, (ROWS_BLK, 1), 0)
    x = jnp.where(row < N, x_ref[...], 0.0)
    dn = (((1,), (1,)), ((), ()))
    fa_ref[...] = lax.dot_general(x, wa_ref[...], dn,
                                  preferred_element_type=jnp.float32
                                  ).astype(jnp.bfloat16)
    fb_ref[...] = lax.dot_general(x, wb_ref[...], dn,
                                  preferred_element_type=jnp.float32
                                  ).astype(jnp.bfloat16)


def _tc_project(features, wa, wb):
    return pl.pallas_call(
        _project_body,
        grid=(NPAD // ROWS_BLK,),
        in_specs=[
            pl.BlockSpec((ROWS_BLK, D), lambda i: (i, 0)),
            pl.BlockSpec((D, D), lambda i: (0, 0)),
            pl.BlockSpec((D, D), lambda i: (0, 0)),
        ],
        out_specs=[
            pl.BlockSpec((ROWS_BLK, D), lambda i: (i, 0)),
            pl.BlockSpec((ROWS_BLK, D), lambda i: (i, 0)),
        ],
        out_shape=[
            jax.ShapeDtypeStruct((NPAD, D), jnp.bfloat16),
            jax.ShapeDtypeStruct((NPAD, D), jnp.bfloat16),
        ],
    )(features, wa, wb)


@functools.cache
def _build_sc_kernel():
    mesh = plsc.VectorSubcoreMesh(core_axis_name="c", subcore_axis_name="s")

    @functools.partial(
        pl.kernel,
        mesh=mesh,
        out_type=jax.ShapeDtypeStruct((B, D), jnp.float32),
        compiler_params=pltpu.CompilerParams(needs_layout_passes=False, use_tc_tiling_on_sc=False),
        scratch_types=[
            pltpu.VMEM((BPW,), jnp.int32),
            pltpu.VMEM((BPW, MAX_LEN), jnp.int32),
            pltpu.VMEM((BPW, MAX_LEN), jnp.float32),
            pltpu.VMEM((NCHUNKS, 128), jnp.int32),
            pltpu.VMEM((BPW,), jnp.float32),
            pltpu.VMEM((BPW, D), jnp.bfloat16),
            pltpu.VMEM((128, D), jnp.bfloat16),
            pltpu.VMEM((128, D), jnp.bfloat16),
            pltpu.VMEM((BPW, D), jnp.float32),
            pltpu.VMEM((D,), jnp.float32),
            pltpu.VMEM_SHARED((NPAD, D), jnp.bfloat16),
            pltpu.SemaphoreType.DMA,
            pltpu.SemaphoreType.DMA,
            pltpu.SemaphoreType.DMA,
        ],
    )
    def _sc_gather_agg(nodes_h, adj_h, mask_h, fa_h, fb_h, b1_h, out_h,
                       nodes_v, adj_v, mask_v, nidx, rden, selfr, nf, nf2, out_v,
                       b1_v, fbs, sem, sem2, sem3):
        wid = lax.axis_index("s") * 2 + lax.axis_index("c")
        base = wid * BPW
        pltpu.sync_copy(nodes_h.at[pl.ds(base, BPW)], nodes_v)
        pltpu.sync_copy(b1_h, b1_v)
        a_cp = pltpu.async_copy(adj_h.at[nodes_v], adj_v, sem)
        m_cp = pltpu.async_copy(mask_h.at[nodes_v], mask_v, sem)
        sid = lax.axis_index("s")
        srow = sid * (NPAD // 16)
        stg_b = pltpu.async_copy(fb_h.at[pl.ds(srow, NPAD // 16)],
                                 fbs.at[pl.ds(srow, NPAD // 16)], sem3)
        s_cp = pltpu.async_copy(fa_h.at[nodes_v], selfr, sem2)
        a_cp.wait()
        m_cp.wait()

        def prep_body(r, carry):
            m0 = mask_v[r, pl.ds(0, 16)]
            m1 = mask_v[r, pl.ds(16, 16)]
            a0 = adj_v[r, pl.ds(0, 16)]
            a1 = adj_v[r, pl.ds(16, 16)]
            pad = jnp.int32(N) + lax.rem(r, 224).astype(jnp.int32)
            i0 = jnp.where(m0 > 0.0, a0, pad)
            i1 = jnp.where(m1 > 0.0, a1, pad)
            p = r * MAX_LEN
            nidx[p // 128, pl.ds(lax.rem(p, 128), 16)] = i0
            nidx[p // 128, pl.ds(lax.rem(p, 128) + 16, 16)] = i1
            return carry

        lax.fori_loop(0, BPW, prep_body, 0)

        lanes = lax.iota(jnp.int32, 16)

        def den_body(g, carry):
            rows = g * 16 + lanes

            def col_body(j, den):
                jj = jnp.full((16,), 0, jnp.int32) + j
                return den + plsc.load_gather(mask_v, [rows, jj])

            den = lax.fori_loop(0, MAX_LEN, col_body,
                                jnp.zeros((16,), jnp.float32))
            rden[pl.ds(g * 16, 16)] = 1.0 / jnp.maximum(den, 1.0)
            return carry

        lax.fori_loop(0, BPW // 16, den_body, 0)
        s_cp.wait()
        stg_b.wait()
        plsc.subcore_barrier()

        def accumulate(c, buf):
            def node_body(n, carry2):
                node = c * CHUNK_NODES + n
                nn = jnp.full((16,), 0, jnp.int32) + node
                rd = plsc.load_gather(rden, [nn])

                def d_body(k, carry3):
                    col = k * 32
                    acc_e = jnp.zeros((16,), jnp.float32)
                    acc_o = jnp.zeros((16,), jnp.float32)
                    for j in range(MAX_LEN):
                        v = buf[n * MAX_LEN + j, pl.ds(col, 32)]
                        e, o = plsc.unpack(
                            v, format=plsc.PackFormat.INTERLEAVED,
                            preferred_element_type=jnp.float32)
                        acc_e = acc_e + e
                        acc_o = acc_o + o
                    sv = selfr[node, pl.ds(col, 32)]
                    se, so = plsc.unpack(
                        sv, format=plsc.PackFormat.INTERLEAVED,
                        preferred_element_type=jnp.float32)
                    for acc, s, cb in ((acc_e, se, col), (acc_o, so, col + 16)):
                        res = s + acc * rd + b1_v[pl.ds(cb, 16)]
                        out_v[node, pl.ds(cb, 16)] = jnp.maximum(res, 0.0)
                    return carry3

                lax.fori_loop(0, D // 32, d_body, 0)
                return carry2

            lax.fori_loop(0, CHUNK_NODES, node_body, 0)

        pltpu.async_copy(fbs.at[nidx.at[0]], nf, sem)
        npairs = NCHUNKS // 2

        def pair_body(g, carry):
            c0 = 2 * g
            pltpu.make_async_copy(fbs.at[nidx.at[c0]], nf, sem).wait()
            cp1 = pltpu.async_copy(fbs.at[nidx.at[c0 + 1]], nf2, sem2)
            accumulate(c0, nf)
            cp1.wait()

            @pl.when(g < npairs - 1)
            def _():
                pltpu.async_copy(fbs.at[nidx.at[c0 + 2]], nf, sem)

            accumulate(c0 + 1, nf2)
            return carry

        lax.fori_loop(0, npairs, pair_body, 0)
        pltpu.sync_copy(out_v, out_h.at[pl.ds(base, BPW)])

    return _sc_gather_agg


import numpy as _np

_m = _np.arange(D)
_COLOF = 32 * (_m // 32) + 16 * (_m % 2) + (_m % 32) // 2


def kernel(nodes, adj, mask, features, W1, b1):
    wa = W1[:, :D][_COLOF]
    wb = W1[:, D:][_COLOF]
    fa, fb = _tc_project(features, wa, wb)
    return _build_sc_kernel()(nodes, adj, mask, fa, fb, b1)
